# baseline (device time: 69010 ns/iter reference)
import jax
import jax.numpy as jnp
from jax import lax
from jax.experimental import pallas as pl
from jax.experimental.pallas import tpu as pltpu

N_DEV = 16


def kernel(x, w_mat):
    m_total, k_shard = x.shape
    k_total, n_total = w_mat.shape
    m_blk = m_total // N_DEV

    def body(x_ref, w_ref, out_ref, xbf_ref, xg_ref, send_sems, recv_sems):
        t = pl.program_id(0)
        my = lax.axis_index("i")

        @pl.when(t == 0)
        def _():
            xbf_ref[...] = x_ref[...].astype(jnp.bfloat16)
            xg_ref[my] = xbf_ref[pl.ds(my * m_blk, m_blk), :]

        for j in range(N_DEV):
            rdma = pltpu.make_async_remote_copy(
                src_ref=xbf_ref.at[pl.ds(j * m_blk, m_blk), :],
                dst_ref=xg_ref.at[my],
                send_sem=send_sems.at[j],
                recv_sem=recv_sems.at[my],
                device_id=(j,),
                device_id_type=pl.DeviceIdType.MESH,
            )

            @pl.when((t == 0) & (j != my))
            def _():
                rdma.start()

            @pl.when((t == N_DEV - 1) & (j != my))
            def _():
                rdma.wait_send()

        recv = pltpu.make_async_remote_copy(
            src_ref=xg_ref.at[t],
            dst_ref=xg_ref.at[t],
            send_sem=send_sems.at[t],
            recv_sem=recv_sems.at[t],
            device_id=(my,),
            device_id_type=pl.DeviceIdType.MESH,
        )

        @pl.when(t != my)
        def _():
            recv.wait_recv()

        prod = jnp.dot(
            xg_ref[t],
            w_ref[...].astype(jnp.bfloat16),
            preferred_element_type=jnp.float32,
        )

        @pl.when(t == 0)
        def _():
            out_ref[...] = prod

        @pl.when(t != 0)
        def _():
            out_ref[...] = out_ref[...] + prod

        @pl.when(t == N_DEV - 1)
        def _():
            out_ref[...] = jnp.maximum(out_ref[...], 0.0)

    return pl.pallas_call(
        body,
        grid=(N_DEV,),
        out_shape=jax.ShapeDtypeStruct((m_blk, n_total), jnp.float32),
        in_specs=[
            pl.BlockSpec((m_total, k_shard), lambda t: (0, 0)),
            pl.BlockSpec((k_total // N_DEV, n_total), lambda t: (t, 0)),
        ],
        out_specs=pl.BlockSpec((m_blk, n_total), lambda t: (0, 0)),
        scratch_shapes=[
            pltpu.VMEM((m_total, k_shard), jnp.bfloat16),
            pltpu.VMEM((N_DEV, m_blk, k_shard), jnp.bfloat16),
            pltpu.SemaphoreType.DMA((N_DEV,)),
            pltpu.SemaphoreType.DMA((N_DEV,)),
        ],
        compiler_params=pltpu.CompilerParams(
            dimension_semantics=("arbitrary",),
        ),
    )(x, w_mat)
